# baseline (speedup 1.0000x reference)
import jax
import jax.numpy as jnp
from jax.experimental import pallas as pl


def _bias_body(h_ref, b_ref, o_ref):
    o_ref[...] = h_ref[...] + b_ref[0][None, :]


def _bias_add(h, b):
    n, d = h.shape
    blk = 1000
    return pl.pallas_call(
        _bias_body,
        out_shape=jax.ShapeDtypeStruct((n, d), h.dtype),
        grid=(n // blk,),
        in_specs=[
            pl.BlockSpec((blk, d), lambda i: (i, 0)),
            pl.BlockSpec((1, d), lambda i: (0, 0)),
        ],
        out_specs=pl.BlockSpec((blk, d), lambda i: (i, 0)),
    )(h, b[None, :])


def _gcn(x, src, dst, ew, W, b):
    n = x.shape[0]
    loop = jnp.arange(n, dtype=src.dtype)
    src2 = jnp.concatenate([src, loop])
    dst2 = jnp.concatenate([dst, loop])
    ew2 = jnp.concatenate([ew, jnp.ones((n,), x.dtype)])
    deg = jnp.zeros((n,), x.dtype).at[dst2].add(ew2)
    dinv = jnp.where(deg > 0, jax.lax.rsqrt(jnp.maximum(deg, 1e-12)), 0.0)
    norm = dinv[src2] * ew2 * dinv[dst2]
    h = x @ W
    msg = h[src2] * norm[:, None]
    out = jnp.zeros((n, W.shape[1]), x.dtype).at[dst2].add(msg)
    return _bias_add(out, b)


def kernel(x, edge_index, edge_weight, W1, b1, W2, b2, W_mu, b_mu, W_ls, b_ls):
    src, dst = edge_index[0], edge_index[1]
    h = jax.nn.relu(_gcn(x, src, dst, edge_weight, W1, b1))
    h = _gcn(h, src, dst, edge_weight, W2, b2)
    ew1 = jnp.ones((edge_weight.shape[0],), x.dtype)
    mu = _gcn(h, src, dst, ew1, W_mu, b_mu)
    logstd = _gcn(h, src, dst, ew1, W_ls, b_ls)
    return (mu, logstd)


# speedup vs baseline: 4.3776x; 4.3776x over previous
import functools

import jax
import jax.numpy as jnp
from jax import lax
from jax.experimental import pallas as pl
from jax.experimental.pallas import tpu as pltpu
from jax.experimental.pallas import tpu_sc as plsc

N = 10000
E = 320000
NP = 10240
NPH = NP // 2
C = 128
NCH = 2560
E2 = NCH * C
L = 16
NT = 16
ROWS_T = NCH // NT
ROWS_W = NCH // 32
NODES_T = NPH // NT
COLS_T = NP // NT

_MESH = plsc.VectorSubcoreMesh(core_axis_name="c", subcore_axis_name="s")
_f32 = jnp.float32

_SC_CP = pltpu.CompilerParams()
if "needs_layout_passes" in pltpu.CompilerParams.__dataclass_fields__:
    import dataclasses as _dc
    _SC_CP = _dc.replace(_SC_CP, needs_layout_passes=False)


def _deg_body(dst_hbm, ew_hbm, degw_hbm, deg1_hbm,
              dst_v, wv, hw, h1, sh, row_v, acc_v):
    c = lax.axis_index("c")
    s = lax.axis_index("s")
    wid = c * NT + s
    r0 = wid * ROWS_W
    pltpu.sync_copy(dst_hbm.at[pl.ds(r0, ROWS_W)], dst_v)
    pltpu.sync_copy(ew_hbm.at[pl.ds(r0, ROWS_W)], wv)
    z = jnp.zeros((L,), _f32)
    ones = jnp.ones((L,), _f32)

    @pl.loop(0, NP, step=L)
    def _(i):
        hw[pl.ds(i, L)] = z
        h1[pl.ds(i, L)] = z

    @pl.loop(0, ROWS_W)
    def _(j):
        for k in range(C // L):
            sl = pl.ds(k * L, L)
            idx = dst_v[j, sl]
            plsc.addupdate_scatter(hw, [idx], wv[j, sl])
            eid = (r0 + j) * C + k * L
            valid = (lax.iota(jnp.int32, L) + eid) < E
            plsc.addupdate_scatter(h1, [idx], ones, mask=valid)

    col0 = s * COLS_T
    for hist, out_hbm in ((hw, degw_hbm), (h1, deg1_hbm)):
        pltpu.sync_copy(hist, sh.at[s])
        plsc.subcore_barrier()

        @pl.loop(0, COLS_T, step=L)
        def _(i):
            acc_v[pl.ds(i, L)] = z

        @pl.loop(0, NT)
        def _(k):
            pltpu.sync_copy(sh.at[k, pl.ds(col0, COLS_T)], row_v)

            @pl.loop(0, COLS_T, step=L)
            def _(i):
                acc_v[pl.ds(i, L)] = acc_v[pl.ds(i, L)] + row_v[pl.ds(i, L)]

        pltpu.sync_copy(acc_v, out_hbm.at[c, pl.ds(col0, COLS_T)])
        plsc.subcore_barrier()


@jax.jit
def _deg_call(dst2d, ew2d):
    f = pl.kernel(
        _deg_body,
        out_type=[jax.ShapeDtypeStruct((2, NP), _f32),
                  jax.ShapeDtypeStruct((2, NP), _f32)],
        mesh=_MESH,
        compiler_params=_SC_CP,
        scratch_types=[
            pltpu.VMEM((ROWS_W, C), jnp.int32),
            pltpu.VMEM((ROWS_W, C), _f32),
            pltpu.VMEM((NP,), _f32),
            pltpu.VMEM((NP,), _f32),
            pltpu.VMEM_SHARED((NT, NP), _f32),
            pltpu.VMEM((COLS_T,), _f32),
            pltpu.VMEM((COLS_T,), _f32),
        ],
    )
    return f(dst2d, ew2d)


def _coef_body(src_hbm, dst_hbm, ew_hbm, dw_hbm, d1_hbm, cw_hbm, c1_hbm,
               src_v, dst_v, ew_v, tw, t1, ow, o1):
    c = lax.axis_index("c")
    s = lax.axis_index("s")
    wid = c * NT + s
    r0 = wid * ROWS_W
    pltpu.sync_copy(src_hbm.at[pl.ds(r0, ROWS_W)], src_v)
    pltpu.sync_copy(dst_hbm.at[pl.ds(r0, ROWS_W)], dst_v)
    pltpu.sync_copy(ew_hbm.at[pl.ds(r0, ROWS_W)], ew_v)
    pltpu.sync_copy(dw_hbm, tw)
    pltpu.sync_copy(d1_hbm, t1)

    @pl.loop(0, ROWS_W)
    def _(j):
        for k in range(C // L):
            sl = pl.ds(k * L, L)
            si = src_v[j, sl]
            di = dst_v[j, sl]
            ow[j, sl] = (plsc.load_gather(tw, [si]) *
                         plsc.load_gather(tw, [di]) * ew_v[j, sl])
            eid = (r0 + j) * C + k * L
            valid = (lax.iota(jnp.int32, L) + eid) < E
            c1 = plsc.load_gather(t1, [si]) * plsc.load_gather(t1, [di])
            o1[j, sl] = jnp.where(valid, c1, 0.0)

    pltpu.sync_copy(ow, cw_hbm.at[pl.ds(r0, ROWS_W)])
    pltpu.sync_copy(o1, c1_hbm.at[pl.ds(r0, ROWS_W)])


@jax.jit
def _coef_call(src2d, dst2d, ew2d, dinv_w, dinv_1):
    f = pl.kernel(
        _coef_body,
        out_type=[jax.ShapeDtypeStruct((NCH, C), _f32),
                  jax.ShapeDtypeStruct((NCH, C), _f32)],
        mesh=_MESH,
        compiler_params=_SC_CP,
        scratch_types=[
            pltpu.VMEM((ROWS_W, C), jnp.int32),
            pltpu.VMEM((ROWS_W, C), jnp.int32),
            pltpu.VMEM((ROWS_W, C), _f32),
            pltpu.VMEM((NP,), _f32),
            pltpu.VMEM((NP,), _f32),
            pltpu.VMEM((ROWS_W, C), _f32),
            pltpu.VMEM((ROWS_W, C), _f32),
        ],
    )
    return f(src2d, dst2d, ew2d, dinv_w, dinv_1)


def _agg_body(p0, stride, fa, fb, sla, slb, src_hbm, dst_hbm, cf_hbm, out_hbm,
              src_v, dst_v, cf_v, rows, acc, sem):
    c = lax.axis_index("c")
    s = lax.axis_index("s")
    base = (p0 + c * stride) * NPH
    r0 = s * ROWS_T
    n0 = s * NODES_T
    pltpu.sync_copy(src_hbm.at[pl.ds(r0, ROWS_T)], src_v)
    pltpu.sync_copy(dst_hbm.at[pl.ds(r0, ROWS_T)], dst_v)
    pltpu.sync_copy(cf_hbm.at[pl.ds(r0, ROWS_T)], cf_v)

    @pl.loop(0, ROWS_T)
    def _(j):
        for k in range(C // L):
            sl = pl.ds(k * L, L)
            ld = dst_v[j, sl] - base
            m = (ld >= 0) & (ld < NPH)
            dst_v[j, sl] = jnp.where(m, ld, 0)
            cf_v[j, sl] = jnp.where(m, cf_v[j, sl], 0.0)

    @pl.when(c == 0)
    def _():
        pltpu.sync_copy(sla.at[pl.ds(n0, NODES_T)], acc.at[pl.ds(n0, NODES_T)])

    @pl.when(c == 1)
    def _():
        pltpu.sync_copy(slb.at[pl.ds(n0, NODES_T)], acc.at[pl.ds(n0, NODES_T)])

    plsc.subcore_barrier()

    def process(feat_hbm):
        @pl.loop(0, ROWS_T)
        def _(j):
            pltpu.async_copy(feat_hbm.at[src_v.at[j]], rows, sem).wait()

            @pl.loop(0, C // L)
            def _(g):
                cvec = cf_v[j, pl.ds(g * L, L)]
                for l in range(L):
                    sc = cvec[l]
                    i = g * L + l
                    for k in range(128 // L):
                        sl = pl.ds(k * L, L)
                        rows[i, sl] = rows[i, sl] * sc

            pltpu.sync_copy(rows, acc.at[dst_v.at[j]], add=True)

    @pl.when(c == 0)
    def _():
        process(fa)

    @pl.when(c == 1)
    def _():
        process(fb)

    plsc.subcore_barrier()
    pltpu.sync_copy(acc.at[pl.ds(n0, NODES_T)], out_hbm.at[c, pl.ds(n0, NODES_T)])


@functools.partial(jax.jit, static_argnums=(0, 1))
def _agg_call(p0, stride, fa, fb, sla, slb, src2d, dst2d, cf2d):
    f = pl.kernel(
        functools.partial(_agg_body, p0, stride),
        out_type=jax.ShapeDtypeStruct((2, NPH, 128), _f32),
        mesh=_MESH,
        scratch_types=[
            pltpu.VMEM((ROWS_T, C), jnp.int32),
            pltpu.VMEM((ROWS_T, C), jnp.int32),
            pltpu.VMEM((ROWS_T, C), _f32),
            pltpu.VMEM((C, 128), _f32),
            pltpu.VMEM_SHARED((NPH, 128), _f32),
            pltpu.SemaphoreType.DMA,
        ],
    )
    return f(fa, fb, sla, slb, src2d, dst2d, cf2d)


_BLK = 1000


def _tc1_body(agg_ref, w_ref, b_ref, dw2_ref, h_ref, sl_ref):
    u = agg_ref[...]
    h = jnp.maximum(jnp.dot(u, w_ref[...],
                            preferred_element_type=_f32) + b_ref[0][None, :], 0.0)
    w2 = dw2_ref[...]
    h_ref[0] = h[:, :128]
    h_ref[1] = h[:, 128:]
    sl_ref[0] = h[:, :128] * w2
    sl_ref[1] = h[:, 128:] * w2


@jax.jit
def _tc1_call(agg1, W1, b1, dw2):
    b1 = b1[None, :]
    return pl.pallas_call(
        _tc1_body,
        out_shape=[jax.ShapeDtypeStruct((2, N, 128), _f32),
                   jax.ShapeDtypeStruct((2, N, 128), _f32)],
        grid=(N // _BLK,),
        in_specs=[
            pl.BlockSpec((_BLK, 128), lambda i: (i, 0)),
            pl.BlockSpec((128, 256), lambda i: (0, 0)),
            pl.BlockSpec((1, 256), lambda i: (0, 0)),
            pl.BlockSpec((_BLK, 1), lambda i: (i, 0)),
        ],
        out_specs=[pl.BlockSpec((2, _BLK, 128), lambda i: (0, i, 0)),
                   pl.BlockSpec((2, _BLK, 128), lambda i: (0, i, 0))],
    )(agg1, W1, b1, dw2)


def _tc2_body(agg_ref, w_ref, b_ref, d12_ref, h_ref, sl_ref):
    h = (jnp.dot(agg_ref[0], w_ref[0], preferred_element_type=_f32) +
         jnp.dot(agg_ref[1], w_ref[1], preferred_element_type=_f32) +
         b_ref[0][None, :])
    w2 = d12_ref[...]
    h_ref[0] = h[:, :128]
    h_ref[1] = h[:, 128:]
    sl_ref[0] = h[:, :128] * w2
    sl_ref[1] = h[:, 128:] * w2


@jax.jit
def _tc2_call(agg2, W2s, b2, d12):
    b2 = b2[None, :]
    return pl.pallas_call(
        _tc2_body,
        out_shape=[jax.ShapeDtypeStruct((2, N, 128), _f32),
                   jax.ShapeDtypeStruct((2, N, 128), _f32)],
        grid=(N // _BLK,),
        in_specs=[
            pl.BlockSpec((2, _BLK, 128), lambda i: (0, i, 0)),
            pl.BlockSpec((2, 128, 256), lambda i: (0, 0, 0)),
            pl.BlockSpec((1, 256), lambda i: (0, 0)),
            pl.BlockSpec((_BLK, 1), lambda i: (i, 0)),
        ],
        out_specs=[pl.BlockSpec((2, _BLK, 128), lambda i: (0, i, 0)),
                   pl.BlockSpec((2, _BLK, 128), lambda i: (0, i, 0))],
    )(agg2, W2s, b2, d12)


def _tc3_body(agg_ref, wmu_ref, bmu_ref, wls_ref, bls_ref, mu_ref, ls_ref):
    a0 = agg_ref[0]
    a1 = agg_ref[1]
    mu_ref[...] = (jnp.dot(a0, wmu_ref[0], preferred_element_type=_f32) +
                   jnp.dot(a1, wmu_ref[1], preferred_element_type=_f32) +
                   bmu_ref[0][None, :])
    ls_ref[...] = (jnp.dot(a0, wls_ref[0], preferred_element_type=_f32) +
                   jnp.dot(a1, wls_ref[1], preferred_element_type=_f32) +
                   bls_ref[0][None, :])


@jax.jit
def _tc3_call(agg3, Wmus, bmu, Wlss, bls):
    bmu = bmu[None, :]
    bls = bls[None, :]
    return pl.pallas_call(
        _tc3_body,
        out_shape=[jax.ShapeDtypeStruct((N, 128), _f32),
                   jax.ShapeDtypeStruct((N, 128), _f32)],
        grid=(N // _BLK,),
        in_specs=[
            pl.BlockSpec((2, _BLK, 128), lambda i: (0, i, 0)),
            pl.BlockSpec((2, 128, 128), lambda i: (0, 0, 0)),
            pl.BlockSpec((1, 128), lambda i: (0, 0)),
            pl.BlockSpec((2, 128, 128), lambda i: (0, 0, 0)),
            pl.BlockSpec((1, 128), lambda i: (0, 0)),
        ],
        out_specs=[pl.BlockSpec((_BLK, 128), lambda i: (i, 0)),
                   pl.BlockSpec((_BLK, 128), lambda i: (i, 0))],
    )(agg3, Wmus, bmu, Wlss, bls)


def kernel(x, edge_index, edge_weight, W1, b1, W2, b2, W_mu, b_mu, W_ls, b_ls):
    src = edge_index[0]
    dst = edge_index[1]
    padn = E2 - E
    zi = jnp.zeros((padn,), jnp.int32)
    src2d = jnp.concatenate([src, zi]).reshape(NCH, C)
    dst2d = jnp.concatenate([dst, zi]).reshape(NCH, C)
    ew2d = jnp.concatenate([edge_weight, jnp.zeros((padn,), _f32)]).reshape(NCH, C)

    degw_p, deg1_p = _deg_call(dst2d, ew2d)
    deg_w = degw_p[0] + degw_p[1] + 1.0
    deg_1 = deg1_p[0] + deg1_p[1] + 1.0
    dinv_w = lax.rsqrt(deg_w)
    dinv_1 = lax.rsqrt(deg_1)

    cw2d, c12d = _coef_call(src2d, dst2d, ew2d, dinv_w, dinv_1)

    dw2 = (dinv_w[:N] ** 2)[:, None]
    d12 = (dinv_1[:N] ** 2)[:, None]

    def padrows(a):
        return jnp.concatenate([a, jnp.zeros((NP - N, a.shape[1]), a.dtype)])

    sl1 = padrows(dw2 * x)
    o1 = _agg_call(0, 1, x, x, sl1[:NPH], sl1[NPH:], src2d, dst2d, cw2d)
    agg1 = jnp.concatenate([o1[0], o1[1]])[:N]
    h1, sl2 = _tc1_call(agg1, W1, b1, dw2)

    def agg_256(fh, slh, cf2d):
        sa = padrows(slh[0])
        sb = padrows(slh[1])
        op0 = _agg_call(0, 0, fh[0], fh[1], sa[:NPH], sb[:NPH],
                        src2d, dst2d, cf2d)
        op1 = _agg_call(1, 0, fh[0], fh[1], sa[NPH:], sb[NPH:],
                        src2d, dst2d, cf2d)
        return jnp.stack([jnp.concatenate([op0[0], op1[0]])[:N],
                          jnp.concatenate([op0[1], op1[1]])[:N]])

    agg2 = agg_256(h1, sl2, cw2d)
    W2s = jnp.stack([W2[:128], W2[128:]])
    h2, sl3 = _tc2_call(agg2, W2s, b2, d12)
    agg3 = agg_256(h2, sl3, c12d)
    Wmus = jnp.stack([W_mu[:128], W_mu[128:]])
    Wlss = jnp.stack([W_ls[:128], W_ls[128:]])
    mu, logstd = _tc3_call(agg3, Wmus, b_mu, Wlss, b_ls)
    return (mu, logstd)


# speedup vs baseline: 4.3982x; 1.0047x over previous
import functools

import jax
import jax.numpy as jnp
from jax import lax
from jax.experimental import pallas as pl
from jax.experimental.pallas import tpu as pltpu
from jax.experimental.pallas import tpu_sc as plsc

N = 10000
E = 320000
NP = 10240
NPH = NP // 2
C = 128
NCH = 2560
E2 = NCH * C
L = 16
NT = 16
ROWS_T = NCH // NT
ROWS_W = NCH // 32
NODES_T = NPH // NT
COLS_T = NP // NT

_MESH = plsc.VectorSubcoreMesh(core_axis_name="c", subcore_axis_name="s")
_f32 = jnp.float32

_SC_CP = pltpu.CompilerParams()
if "needs_layout_passes" in pltpu.CompilerParams.__dataclass_fields__:
    import dataclasses as _dc
    _SC_CP = _dc.replace(_SC_CP, needs_layout_passes=False)


_HR = NP // C


def _deg_body(dst_hbm, ew_hbm, degw_hbm, deg1_hbm,
              dst_v, wv, hw, h1, idx_v, zrow, shw, sh1):
    c = lax.axis_index("c")
    s = lax.axis_index("s")
    wid = c * NT + s
    r0 = wid * ROWS_W
    pltpu.sync_copy(dst_hbm.at[pl.ds(r0, ROWS_W)], dst_v)
    pltpu.sync_copy(ew_hbm.at[pl.ds(r0, ROWS_W)], wv)
    z = jnp.zeros((L,), _f32)
    ones = jnp.ones((L,), _f32)

    @pl.loop(0, _HR)
    def _(j):
        for k in range(C // L):
            sl = pl.ds(k * L, L)
            hw[j, sl] = z
            h1[j, sl] = z

    @pl.loop(0, ROWS_W)
    def _(j):
        for k in range(C // L):
            sl = pl.ds(k * L, L)
            idx = dst_v[j, sl]
            hi = lax.shift_right_logical(idx, 7)
            lo = lax.bitwise_and(idx, 127)
            plsc.addupdate_scatter(hw, [hi, lo], wv[j, sl])
            eid = (r0 + j) * C + k * L
            valid = (lax.iota(jnp.int32, L) + eid) < E
            plsc.addupdate_scatter(h1, [hi, lo], ones, mask=valid)

    @pl.loop(0, 5)
    def _(j):
        for k in range(C // L):
            zrow[j, pl.ds(k * L, L)] = z

    @pl.loop(0, _HR, step=L)
    def _(i):
        idx_v[pl.ds(i, L)] = lax.iota(jnp.int32, L) + i

    a0 = s * 5
    pltpu.sync_copy(zrow, shw.at[pl.ds(a0, 5)])
    pltpu.sync_copy(zrow, sh1.at[pl.ds(a0, 5)])
    plsc.subcore_barrier()
    pltpu.sync_copy(hw, shw.at[idx_v], add=True)
    pltpu.sync_copy(h1, sh1.at[idx_v], add=True)
    plsc.subcore_barrier()

    @pl.when(s < 10)
    def _():
        b0 = s * 8
        pltpu.sync_copy(shw.at[pl.ds(b0, 8)], degw_hbm.at[c, pl.ds(b0, 8)])
        pltpu.sync_copy(sh1.at[pl.ds(b0, 8)], deg1_hbm.at[c, pl.ds(b0, 8)])


@jax.jit
def _deg_call(dst2d, ew2d):
    f = pl.kernel(
        _deg_body,
        out_type=[jax.ShapeDtypeStruct((2, _HR, C), _f32),
                  jax.ShapeDtypeStruct((2, _HR, C), _f32)],
        mesh=_MESH,
        compiler_params=_SC_CP,
        scratch_types=[
            pltpu.VMEM((ROWS_W, C), jnp.int32),
            pltpu.VMEM((ROWS_W, C), _f32),
            pltpu.VMEM((_HR, C), _f32),
            pltpu.VMEM((_HR, C), _f32),
            pltpu.VMEM((_HR,), jnp.int32),
            pltpu.VMEM((5, C), _f32),
            pltpu.VMEM_SHARED((_HR, C), _f32),
            pltpu.VMEM_SHARED((_HR, C), _f32),
        ],
    )
    return f(dst2d, ew2d)


def _coef_body(src_hbm, dst_hbm, ew_hbm, dw_hbm, d1_hbm, cw_hbm, c1_hbm,
               src_v, dst_v, ew_v, tw, t1, ow, o1):
    c = lax.axis_index("c")
    s = lax.axis_index("s")
    wid = c * NT + s
    r0 = wid * ROWS_W
    pltpu.sync_copy(src_hbm.at[pl.ds(r0, ROWS_W)], src_v)
    pltpu.sync_copy(dst_hbm.at[pl.ds(r0, ROWS_W)], dst_v)
    pltpu.sync_copy(ew_hbm.at[pl.ds(r0, ROWS_W)], ew_v)
    pltpu.sync_copy(dw_hbm, tw)
    pltpu.sync_copy(d1_hbm, t1)

    @pl.loop(0, ROWS_W)
    def _(j):
        for k in range(C // L):
            sl = pl.ds(k * L, L)
            si = src_v[j, sl]
            di = dst_v[j, sl]
            ow[j, sl] = (plsc.load_gather(tw, [si]) *
                         plsc.load_gather(tw, [di]) * ew_v[j, sl])
            eid = (r0 + j) * C + k * L
            valid = (lax.iota(jnp.int32, L) + eid) < E
            c1 = plsc.load_gather(t1, [si]) * plsc.load_gather(t1, [di])
            o1[j, sl] = jnp.where(valid, c1, 0.0)

    pltpu.sync_copy(ow, cw_hbm.at[pl.ds(r0, ROWS_W)])
    pltpu.sync_copy(o1, c1_hbm.at[pl.ds(r0, ROWS_W)])


@jax.jit
def _coef_call(src2d, dst2d, ew2d, dinv_w, dinv_1):
    f = pl.kernel(
        _coef_body,
        out_type=[jax.ShapeDtypeStruct((NCH, C), _f32),
                  jax.ShapeDtypeStruct((NCH, C), _f32)],
        mesh=_MESH,
        compiler_params=_SC_CP,
        scratch_types=[
            pltpu.VMEM((ROWS_W, C), jnp.int32),
            pltpu.VMEM((ROWS_W, C), jnp.int32),
            pltpu.VMEM((ROWS_W, C), _f32),
            pltpu.VMEM((NP,), _f32),
            pltpu.VMEM((NP,), _f32),
            pltpu.VMEM((ROWS_W, C), _f32),
            pltpu.VMEM((ROWS_W, C), _f32),
        ],
    )
    return f(src2d, dst2d, ew2d, dinv_w, dinv_1)


def _agg_body(bc, bp, npass, fa, fb, sla, slb, src_hbm, dst_hbm, cf_hbm,
              out_hbm, src_v, dst_v, cf_v, dm, rows0, rows1, rows2, rows3,
              acc, gsem0, gsem1, gsem2, gsem3, ssem):
    c = lax.axis_index("c")
    s = lax.axis_index("s")
    r0 = s * ROWS_T
    n0 = s * NODES_T
    pltpu.sync_copy(src_hbm.at[pl.ds(r0, ROWS_T)], src_v)
    pltpu.sync_copy(dst_hbm.at[pl.ds(r0, ROWS_T)], dst_v)
    pltpu.sync_copy(cf_hbm.at[pl.ds(r0, ROWS_T)], cf_v)

    def process(feat_hbm, base):
        @pl.loop(0, ROWS_T)
        def _(jj):
            pltpu.async_copy(feat_hbm.at[src_v.at[jj]], rows0, gsem0).wait()

            @pl.loop(0, C // L)
            def _(g):
                sl = pl.ds(g * L, L)
                ld = dst_v[jj, sl] - base
                m = (ld >= 0) & (ld < NPH)
                dm[0, sl] = jnp.where(m, ld, 0)
                cvec = jnp.where(m, cf_v[jj, sl], 0.0)
                for l in range(L):
                    sc = cvec[l]
                    i = g * L + l
                    for k in range(128 // L):
                        ksl = pl.ds(k * L, L)
                        rows0[i, ksl] = rows0[i, ksl] * sc

            pltpu.sync_copy(rows0, acc.at[dm.at[0]], add=True)

    for p in range(npass):
        base = (bc * c + bp * p) * NPH

        @pl.when(c == 0)
        def _():
            pltpu.sync_copy(sla.at[pl.ds(base + n0, NODES_T)],
                            acc.at[pl.ds(n0, NODES_T)])

        @pl.when(c == 1)
        def _():
            pltpu.sync_copy(slb.at[pl.ds(base + n0, NODES_T)],
                            acc.at[pl.ds(n0, NODES_T)])

        plsc.subcore_barrier()

        @pl.when(c == 0)
        def _():
            process(fa, base)

        @pl.when(c == 1)
        def _():
            process(fb, base)

        plsc.subcore_barrier()
        pltpu.sync_copy(acc.at[pl.ds(n0, NODES_T)],
                        out_hbm.at[c, pl.ds(base + n0, NODES_T)])


@functools.partial(jax.jit, static_argnums=(0, 1, 2))
def _agg_call(bc, bp, npass, fa, fb, sla, slb, src2d, dst2d, cf2d):
    f = pl.kernel(
        functools.partial(_agg_body, bc, bp, npass),
        out_type=jax.ShapeDtypeStruct((2, NP, 128), _f32),
        mesh=_MESH,
        scratch_types=[
            pltpu.VMEM((ROWS_T, C), jnp.int32),
            pltpu.VMEM((ROWS_T, C), jnp.int32),
            pltpu.VMEM((ROWS_T, C), _f32),
            pltpu.VMEM((4, C), jnp.int32),
            pltpu.VMEM((C, 128), _f32),
            pltpu.VMEM((C, 128), _f32),
            pltpu.VMEM((C, 128), _f32),
            pltpu.VMEM((C, 128), _f32),
            pltpu.VMEM_SHARED((NPH, 128), _f32),
            pltpu.SemaphoreType.DMA,
            pltpu.SemaphoreType.DMA,
            pltpu.SemaphoreType.DMA,
            pltpu.SemaphoreType.DMA,
            pltpu.SemaphoreType.DMA,
        ],
    )
    return f(fa, fb, sla, slb, src2d, dst2d, cf2d)


_BLK = 1000


def _tc1_body(agg_ref, w_ref, b_ref, dw2_ref, h_ref, sl_ref):
    u = agg_ref[...]
    h = jnp.maximum(jnp.dot(u, w_ref[...],
                            preferred_element_type=_f32) + b_ref[0][None, :], 0.0)
    w2 = dw2_ref[...]
    h_ref[0] = h[:, :128]
    h_ref[1] = h[:, 128:]
    sl_ref[0] = h[:, :128] * w2
    sl_ref[1] = h[:, 128:] * w2


@jax.jit
def _tc1_call(agg1, W1, b1, dw2):
    b1 = b1[None, :]
    return pl.pallas_call(
        _tc1_body,
        out_shape=[jax.ShapeDtypeStruct((2, N, 128), _f32),
                   jax.ShapeDtypeStruct((2, N, 128), _f32)],
        grid=(N // _BLK,),
        in_specs=[
            pl.BlockSpec((_BLK, 128), lambda i: (i, 0)),
            pl.BlockSpec((128, 256), lambda i: (0, 0)),
            pl.BlockSpec((1, 256), lambda i: (0, 0)),
            pl.BlockSpec((_BLK, 1), lambda i: (i, 0)),
        ],
        out_specs=[pl.BlockSpec((2, _BLK, 128), lambda i: (0, i, 0)),
                   pl.BlockSpec((2, _BLK, 128), lambda i: (0, i, 0))],
    )(agg1, W1, b1, dw2)


def _tc2_body(agg_ref, w_ref, b_ref, d12_ref, h_ref, sl_ref):
    h = (jnp.dot(agg_ref[0], w_ref[0], preferred_element_type=_f32) +
         jnp.dot(agg_ref[1], w_ref[1], preferred_element_type=_f32) +
         b_ref[0][None, :])
    w2 = d12_ref[...]
    h_ref[0] = h[:, :128]
    h_ref[1] = h[:, 128:]
    sl_ref[0] = h[:, :128] * w2
    sl_ref[1] = h[:, 128:] * w2


@jax.jit
def _tc2_call(agg2, W2s, b2, d12):
    b2 = b2[None, :]
    return pl.pallas_call(
        _tc2_body,
        out_shape=[jax.ShapeDtypeStruct((2, N, 128), _f32),
                   jax.ShapeDtypeStruct((2, N, 128), _f32)],
        grid=(N // _BLK,),
        in_specs=[
            pl.BlockSpec((2, _BLK, 128), lambda i: (0, i, 0)),
            pl.BlockSpec((2, 128, 256), lambda i: (0, 0, 0)),
            pl.BlockSpec((1, 256), lambda i: (0, 0)),
            pl.BlockSpec((_BLK, 1), lambda i: (i, 0)),
        ],
        out_specs=[pl.BlockSpec((2, _BLK, 128), lambda i: (0, i, 0)),
                   pl.BlockSpec((2, _BLK, 128), lambda i: (0, i, 0))],
    )(agg2, W2s, b2, d12)


def _tc3_body(agg_ref, wmu_ref, bmu_ref, wls_ref, bls_ref, mu_ref, ls_ref):
    a0 = agg_ref[0]
    a1 = agg_ref[1]
    mu_ref[...] = (jnp.dot(a0, wmu_ref[0], preferred_element_type=_f32) +
                   jnp.dot(a1, wmu_ref[1], preferred_element_type=_f32) +
                   bmu_ref[0][None, :])
    ls_ref[...] = (jnp.dot(a0, wls_ref[0], preferred_element_type=_f32) +
                   jnp.dot(a1, wls_ref[1], preferred_element_type=_f32) +
                   bls_ref[0][None, :])


@jax.jit
def _tc3_call(agg3, Wmus, bmu, Wlss, bls):
    bmu = bmu[None, :]
    bls = bls[None, :]
    return pl.pallas_call(
        _tc3_body,
        out_shape=[jax.ShapeDtypeStruct((N, 128), _f32),
                   jax.ShapeDtypeStruct((N, 128), _f32)],
        grid=(N // _BLK,),
        in_specs=[
            pl.BlockSpec((2, _BLK, 128), lambda i: (0, i, 0)),
            pl.BlockSpec((2, 128, 128), lambda i: (0, 0, 0)),
            pl.BlockSpec((1, 128), lambda i: (0, 0)),
            pl.BlockSpec((2, 128, 128), lambda i: (0, 0, 0)),
            pl.BlockSpec((1, 128), lambda i: (0, 0)),
        ],
        out_specs=[pl.BlockSpec((_BLK, 128), lambda i: (i, 0)),
                   pl.BlockSpec((_BLK, 128), lambda i: (i, 0))],
    )(agg3, Wmus, bmu, Wlss, bls)


def kernel(x, edge_index, edge_weight, W1, b1, W2, b2, W_mu, b_mu, W_ls, b_ls):
    src = edge_index[0]
    dst = edge_index[1]
    padn = E2 - E
    zi = jnp.zeros((padn,), jnp.int32)
    src2d = jnp.concatenate([src, zi]).reshape(NCH, C)
    dst2d = jnp.concatenate([dst, zi]).reshape(NCH, C)
    ew2d = jnp.concatenate([edge_weight, jnp.zeros((padn,), _f32)]).reshape(NCH, C)

    degw_p, deg1_p = _deg_call(dst2d, ew2d)
    degw_p = degw_p.reshape(2, NP)
    deg1_p = deg1_p.reshape(2, NP)
    deg_w = degw_p[0] + degw_p[1] + 1.0
    deg_1 = deg1_p[0] + deg1_p[1] + 1.0
    dinv_w = lax.rsqrt(deg_w)
    dinv_1 = lax.rsqrt(deg_1)

    cw2d, c12d = _coef_call(src2d, dst2d, ew2d, dinv_w, dinv_1)

    dw2 = (dinv_w[:N] ** 2)[:, None]
    d12 = (dinv_1[:N] ** 2)[:, None]

    def padrows(a):
        return jnp.concatenate([a, jnp.zeros((NP - N, a.shape[1]), a.dtype)])

    sl1 = padrows(dw2 * x)
    o1 = _agg_call(1, 0, 1, x, x, sl1, sl1, src2d, dst2d, cw2d)
    agg1 = jnp.concatenate([o1[0, :NPH], o1[1, NPH:]])[:N]
    h1, sl2 = _tc1_call(agg1, W1, b1, dw2)

    def agg_256(fh, slh, cf2d):
        o = _agg_call(0, 1, 2, fh[0], fh[1], padrows(slh[0]), padrows(slh[1]),
                      src2d, dst2d, cf2d)
        return o[:, :N]

    agg2 = agg_256(h1, sl2, cw2d)
    W2s = jnp.stack([W2[:128], W2[128:]])
    h2, sl3 = _tc2_call(agg2, W2s, b2, d12)
    agg3 = agg_256(h2, sl3, c12d)
    Wmus = jnp.stack([W_mu[:128], W_mu[128:]])
    Wlss = jnp.stack([W_ls[:128], W_ls[128:]])
    mu, logstd = _tc3_call(agg3, Wmus, b_mu, Wlss, b_ls)
    return (mu, logstd)


# speedup vs baseline: 6.6139x; 1.5038x over previous
import functools

import jax
import jax.numpy as jnp
from jax import lax
from jax.experimental import pallas as pl
from jax.experimental.pallas import tpu as pltpu
from jax.experimental.pallas import tpu_sc as plsc

N = 10000
E = 320000
NP = 10240
NPH = NP // 2
C = 128
NCH = 2560
E2 = NCH * C
L = 16
NT = 16
ROWS_T = NCH // NT
ROWS_W = NCH // 32
NODES_T = NPH // NT
COLS_T = NP // NT

_MESH = plsc.VectorSubcoreMesh(core_axis_name="c", subcore_axis_name="s")
_f32 = jnp.float32

_SC_CP = pltpu.CompilerParams()
if "needs_layout_passes" in pltpu.CompilerParams.__dataclass_fields__:
    import dataclasses as _dc
    _SC_CP = _dc.replace(_SC_CP, needs_layout_passes=False)


_HR = NP // C


def _deg_body(dst_hbm, ew_hbm, degw_hbm, deg1_hbm,
              dst_v, wv, hw, h1, idx_v, zrow, shw, sh1):
    c = lax.axis_index("c")
    s = lax.axis_index("s")
    wid = c * NT + s
    r0 = wid * ROWS_W
    pltpu.sync_copy(dst_hbm.at[pl.ds(r0, ROWS_W)], dst_v)
    pltpu.sync_copy(ew_hbm.at[pl.ds(r0, ROWS_W)], wv)
    z = jnp.zeros((L,), _f32)
    ones = jnp.ones((L,), _f32)

    @pl.loop(0, _HR)
    def _(j):
        for k in range(C // L):
            sl = pl.ds(k * L, L)
            hw[j, sl] = z
            h1[j, sl] = z

    @pl.loop(0, ROWS_W)
    def _(j):
        for k in range(C // L):
            sl = pl.ds(k * L, L)
            idx = dst_v[j, sl]
            hi = lax.shift_right_logical(idx, 7)
            lo = lax.bitwise_and(idx, 127)
            plsc.addupdate_scatter(hw, [hi, lo], wv[j, sl])
            eid = (r0 + j) * C + k * L
            valid = (lax.iota(jnp.int32, L) + eid) < E
            plsc.addupdate_scatter(h1, [hi, lo], ones, mask=valid)

    @pl.loop(0, 5)
    def _(j):
        for k in range(C // L):
            zrow[j, pl.ds(k * L, L)] = z

    @pl.loop(0, _HR, step=L)
    def _(i):
        idx_v[pl.ds(i, L)] = lax.iota(jnp.int32, L) + i

    a0 = s * 5
    pltpu.sync_copy(zrow, shw.at[pl.ds(a0, 5)])
    pltpu.sync_copy(zrow, sh1.at[pl.ds(a0, 5)])
    plsc.subcore_barrier()
    pltpu.sync_copy(hw, shw.at[idx_v], add=True)
    pltpu.sync_copy(h1, sh1.at[idx_v], add=True)
    plsc.subcore_barrier()

    @pl.when(s < 10)
    def _():
        b0 = s * 8
        pltpu.sync_copy(shw.at[pl.ds(b0, 8)], degw_hbm.at[c, pl.ds(b0, 8)])
        pltpu.sync_copy(sh1.at[pl.ds(b0, 8)], deg1_hbm.at[c, pl.ds(b0, 8)])


@jax.jit
def _deg_call(dst2d, ew2d):
    f = pl.kernel(
        _deg_body,
        out_type=[jax.ShapeDtypeStruct((2, _HR, C), _f32),
                  jax.ShapeDtypeStruct((2, _HR, C), _f32)],
        mesh=_MESH,
        compiler_params=_SC_CP,
        scratch_types=[
            pltpu.VMEM((ROWS_W, C), jnp.int32),
            pltpu.VMEM((ROWS_W, C), _f32),
            pltpu.VMEM((_HR, C), _f32),
            pltpu.VMEM((_HR, C), _f32),
            pltpu.VMEM((_HR,), jnp.int32),
            pltpu.VMEM((5, C), _f32),
            pltpu.VMEM_SHARED((_HR, C), _f32),
            pltpu.VMEM_SHARED((_HR, C), _f32),
        ],
    )
    return f(dst2d, ew2d)


def _coef_body(src_hbm, dst_hbm, ew_hbm, dw_hbm, d1_hbm, cw_hbm, c1_hbm,
               src_v, dst_v, ew_v, tw, t1, ow, o1):
    c = lax.axis_index("c")
    s = lax.axis_index("s")
    wid = c * NT + s
    r0 = wid * ROWS_W
    pltpu.sync_copy(src_hbm.at[pl.ds(r0, ROWS_W)], src_v)
    pltpu.sync_copy(dst_hbm.at[pl.ds(r0, ROWS_W)], dst_v)
    pltpu.sync_copy(ew_hbm.at[pl.ds(r0, ROWS_W)], ew_v)
    pltpu.sync_copy(dw_hbm, tw)
    pltpu.sync_copy(d1_hbm, t1)

    @pl.loop(0, ROWS_W)
    def _(j):
        for k in range(C // L):
            sl = pl.ds(k * L, L)
            si = src_v[j, sl]
            di = dst_v[j, sl]
            ow[j, sl] = (plsc.load_gather(tw, [si]) *
                         plsc.load_gather(tw, [di]) * ew_v[j, sl])
            eid = (r0 + j) * C + k * L
            valid = (lax.iota(jnp.int32, L) + eid) < E
            c1 = plsc.load_gather(t1, [si]) * plsc.load_gather(t1, [di])
            o1[j, sl] = jnp.where(valid, c1, 0.0)

    pltpu.sync_copy(ow, cw_hbm.at[pl.ds(r0, ROWS_W)])
    pltpu.sync_copy(o1, c1_hbm.at[pl.ds(r0, ROWS_W)])


@jax.jit
def _coef_call(src2d, dst2d, ew2d, dinv_w, dinv_1):
    f = pl.kernel(
        _coef_body,
        out_type=[jax.ShapeDtypeStruct((NCH, C), _f32),
                  jax.ShapeDtypeStruct((NCH, C), _f32)],
        mesh=_MESH,
        compiler_params=_SC_CP,
        scratch_types=[
            pltpu.VMEM((ROWS_W, C), jnp.int32),
            pltpu.VMEM((ROWS_W, C), jnp.int32),
            pltpu.VMEM((ROWS_W, C), _f32),
            pltpu.VMEM((NP,), _f32),
            pltpu.VMEM((NP,), _f32),
            pltpu.VMEM((ROWS_W, C), _f32),
            pltpu.VMEM((ROWS_W, C), _f32),
        ],
    )
    return f(src2d, dst2d, ew2d, dinv_w, dinv_1)


ET = ROWS_T * C


def _agg_body(bc, bp, npass, fa, fb, sla, slb, src_hbm, dst_hbm, cf_hbm,
              out_hbm, src1d, dst1d, cf1d, dm, rows, acc):
    c = lax.axis_index("c")
    s = lax.axis_index("s")
    e0 = s * ET
    n0 = s * NODES_T

    def process(feat_hbm, base):
        pltpu.sync_copy(src_hbm.at[pl.ds(e0, ET)], src1d.at[pl.ds(0, ET)])
        pltpu.sync_copy(dst_hbm.at[pl.ds(e0, ET)], dst1d.at[pl.ds(0, ET)])
        pltpu.sync_copy(cf_hbm.at[pl.ds(e0, ET)], cf1d.at[pl.ds(0, ET)])

        def cbody(i, off):
            sl = pl.ds(i * L, L)
            sv = src1d[sl]
            dv = dst1d[sl] - base
            cv = cf1d[sl]
            m = (dv >= 0) & (dv < NPH)
            plsc.store_compressed(src1d.at[pl.ds(off, L)], sv, mask=m)
            plsc.store_compressed(dst1d.at[pl.ds(off, L)], dv, mask=m)
            plsc.store_compressed(cf1d.at[pl.ds(off, L)], cv, mask=m)
            return off + plsc.all_reduce_population_count(m)[0]

        off = lax.fori_loop(0, ET // L, cbody, jnp.int32(0))

        bt = (off // L) * L
        lm = lax.iota(jnp.int32, L) >= (off - bt)
        for ref in (src1d, dst1d):
            v = ref[pl.ds(bt, L)]
            ref[pl.ds(bt, L)] = jnp.where(lm, 0, v)
        vf = cf1d[pl.ds(bt, L)]
        cf1d[pl.ds(bt, L)] = jnp.where(lm, 0.0, vf)
        for k in range(1, 10):
            src1d[pl.ds(bt + k * L, L)] = jnp.zeros((L,), jnp.int32)
            dst1d[pl.ds(bt + k * L, L)] = jnp.zeros((L,), jnp.int32)
            cf1d[pl.ds(bt + k * L, L)] = jnp.zeros((L,), _f32)

        ncht = (off + C - 1) // C

        @pl.loop(0, ROWS_T)
        def _(qj):
            @pl.when(qj < ncht)
            def _():
                for k in range(C // L):
                    dm[0, pl.ds(k * L, L)] = dst1d[pl.ds(qj * C + k * L, L)]
                pltpu.sync_copy(feat_hbm.at[src1d.at[pl.ds(qj * C, C)]], rows)

                @pl.loop(0, C // L)
                def _(g):
                    cvec = cf1d[pl.ds(qj * C + g * L, L)]
                    for l in range(L):
                        sc = cvec[l]
                        i = g * L + l
                        for k in range(128 // L):
                            ksl = pl.ds(k * L, L)
                            rows[i, ksl] = rows[i, ksl] * sc

                pltpu.sync_copy(rows, acc.at[dm.at[0]], add=True)

    for p in range(npass):
        base = (bc * c + bp * p) * NPH

        @pl.when(c == 0)
        def _():
            pltpu.sync_copy(sla.at[pl.ds(base + n0, NODES_T)],
                            acc.at[pl.ds(n0, NODES_T)])

        @pl.when(c == 1)
        def _():
            pltpu.sync_copy(slb.at[pl.ds(base + n0, NODES_T)],
                            acc.at[pl.ds(n0, NODES_T)])

        plsc.subcore_barrier()

        @pl.when(c == 0)
        def _():
            process(fa, base)

        @pl.when(c == 1)
        def _():
            process(fb, base)

        plsc.subcore_barrier()
        pltpu.sync_copy(acc.at[pl.ds(n0, NODES_T)],
                        out_hbm.at[c, pl.ds(base + n0, NODES_T)])


@functools.partial(jax.jit, static_argnums=(0, 1, 2))
def _agg_call(bc, bp, npass, fa, fb, sla, slb, src1, dst1, cf1):
    f = pl.kernel(
        functools.partial(_agg_body, bc, bp, npass),
        out_type=jax.ShapeDtypeStruct((2, NP, 128), _f32),
        mesh=_MESH,
        compiler_params=_SC_CP,
        scratch_types=[
            pltpu.VMEM((ET + 2 * C,), jnp.int32),
            pltpu.VMEM((ET + 2 * C,), jnp.int32),
            pltpu.VMEM((ET + 2 * C,), _f32),
            pltpu.VMEM((1, C), jnp.int32),
            pltpu.VMEM((C, 128), _f32),
            pltpu.VMEM_SHARED((NPH, 128), _f32),
        ],
    )
    return f(fa, fb, sla, slb, src1, dst1, cf1)


_BLK = 1000


def _tc1_body(agg_ref, w_ref, b_ref, dw2_ref, h_ref, sl_ref):
    u = agg_ref[...]
    h = jnp.maximum(jnp.dot(u, w_ref[...],
                            preferred_element_type=_f32) + b_ref[0][None, :], 0.0)
    w2 = dw2_ref[...]
    h_ref[0] = h[:, :128]
    h_ref[1] = h[:, 128:]
    sl_ref[0] = h[:, :128] * w2
    sl_ref[1] = h[:, 128:] * w2


@jax.jit
def _tc1_call(agg1, W1, b1, dw2):
    b1 = b1[None, :]
    return pl.pallas_call(
        _tc1_body,
        out_shape=[jax.ShapeDtypeStruct((2, N, 128), _f32),
                   jax.ShapeDtypeStruct((2, N, 128), _f32)],
        grid=(N // _BLK,),
        in_specs=[
            pl.BlockSpec((_BLK, 128), lambda i: (i, 0)),
            pl.BlockSpec((128, 256), lambda i: (0, 0)),
            pl.BlockSpec((1, 256), lambda i: (0, 0)),
            pl.BlockSpec((_BLK, 1), lambda i: (i, 0)),
        ],
        out_specs=[pl.BlockSpec((2, _BLK, 128), lambda i: (0, i, 0)),
                   pl.BlockSpec((2, _BLK, 128), lambda i: (0, i, 0))],
    )(agg1, W1, b1, dw2)


def _tc2_body(agg_ref, w_ref, b_ref, d12_ref, h_ref, sl_ref):
    h = (jnp.dot(agg_ref[0], w_ref[0], preferred_element_type=_f32) +
         jnp.dot(agg_ref[1], w_ref[1], preferred_element_type=_f32) +
         b_ref[0][None, :])
    w2 = d12_ref[...]
    h_ref[0] = h[:, :128]
    h_ref[1] = h[:, 128:]
    sl_ref[0] = h[:, :128] * w2
    sl_ref[1] = h[:, 128:] * w2


@jax.jit
def _tc2_call(agg2, W2s, b2, d12):
    b2 = b2[None, :]
    return pl.pallas_call(
        _tc2_body,
        out_shape=[jax.ShapeDtypeStruct((2, N, 128), _f32),
                   jax.ShapeDtypeStruct((2, N, 128), _f32)],
        grid=(N // _BLK,),
        in_specs=[
            pl.BlockSpec((2, _BLK, 128), lambda i: (0, i, 0)),
            pl.BlockSpec((2, 128, 256), lambda i: (0, 0, 0)),
            pl.BlockSpec((1, 256), lambda i: (0, 0)),
            pl.BlockSpec((_BLK, 1), lambda i: (i, 0)),
        ],
        out_specs=[pl.BlockSpec((2, _BLK, 128), lambda i: (0, i, 0)),
                   pl.BlockSpec((2, _BLK, 128), lambda i: (0, i, 0))],
    )(agg2, W2s, b2, d12)


def _tc3_body(agg_ref, wmu_ref, bmu_ref, wls_ref, bls_ref, mu_ref, ls_ref):
    a0 = agg_ref[0]
    a1 = agg_ref[1]
    mu_ref[...] = (jnp.dot(a0, wmu_ref[0], preferred_element_type=_f32) +
                   jnp.dot(a1, wmu_ref[1], preferred_element_type=_f32) +
                   bmu_ref[0][None, :])
    ls_ref[...] = (jnp.dot(a0, wls_ref[0], preferred_element_type=_f32) +
                   jnp.dot(a1, wls_ref[1], preferred_element_type=_f32) +
                   bls_ref[0][None, :])


@jax.jit
def _tc3_call(agg3, Wmus, bmu, Wlss, bls):
    bmu = bmu[None, :]
    bls = bls[None, :]
    return pl.pallas_call(
        _tc3_body,
        out_shape=[jax.ShapeDtypeStruct((N, 128), _f32),
                   jax.ShapeDtypeStruct((N, 128), _f32)],
        grid=(N // _BLK,),
        in_specs=[
            pl.BlockSpec((2, _BLK, 128), lambda i: (0, i, 0)),
            pl.BlockSpec((2, 128, 128), lambda i: (0, 0, 0)),
            pl.BlockSpec((1, 128), lambda i: (0, 0)),
            pl.BlockSpec((2, 128, 128), lambda i: (0, 0, 0)),
            pl.BlockSpec((1, 128), lambda i: (0, 0)),
        ],
        out_specs=[pl.BlockSpec((_BLK, 128), lambda i: (i, 0)),
                   pl.BlockSpec((_BLK, 128), lambda i: (i, 0))],
    )(agg3, Wmus, bmu, Wlss, bls)


def kernel(x, edge_index, edge_weight, W1, b1, W2, b2, W_mu, b_mu, W_ls, b_ls):
    src = edge_index[0]
    dst = edge_index[1]
    padn = E2 - E
    zi = jnp.zeros((padn,), jnp.int32)
    src2d = jnp.concatenate([src, zi]).reshape(NCH, C)
    dst2d = jnp.concatenate([dst, zi]).reshape(NCH, C)
    ew2d = jnp.concatenate([edge_weight, jnp.zeros((padn,), _f32)]).reshape(NCH, C)

    degw_p, deg1_p = _deg_call(dst2d, ew2d)
    degw_p = degw_p.reshape(2, NP)
    deg1_p = deg1_p.reshape(2, NP)
    deg_w = degw_p[0] + degw_p[1] + 1.0
    deg_1 = deg1_p[0] + deg1_p[1] + 1.0
    dinv_w = lax.rsqrt(deg_w)
    dinv_1 = lax.rsqrt(deg_1)

    cw2d, c12d = _coef_call(src2d, dst2d, ew2d, dinv_w, dinv_1)

    dw2 = (dinv_w[:N] ** 2)[:, None]
    d12 = (dinv_1[:N] ** 2)[:, None]

    def padrows(a):
        return jnp.concatenate([a, jnp.zeros((NP - N, a.shape[1]), a.dtype)])

    src1 = src2d.reshape(E2)
    dst1 = dst2d.reshape(E2)
    cw1 = cw2d.reshape(E2)
    c11 = c12d.reshape(E2)

    sl1 = padrows(dw2 * x)
    o1 = _agg_call(1, 0, 1, x, x, sl1, sl1, src1, dst1, cw1)
    agg1 = jnp.concatenate([o1[0, :NPH], o1[1, NPH:]])[:N]
    h1, sl2 = _tc1_call(agg1, W1, b1, dw2)

    def agg_256(fh, slh, cf1):
        o = _agg_call(0, 1, 2, fh[0], fh[1], padrows(slh[0]), padrows(slh[1]),
                      src1, dst1, cf1)
        return o[:, :N]

    agg2 = agg_256(h1, sl2, cw1)
    W2s = jnp.stack([W2[:128], W2[128:]])
    h2, sl3 = _tc2_call(agg2, W2s, b2, d12)
    agg3 = agg_256(h2, sl3, c11)
    Wmus = jnp.stack([W_mu[:128], W_mu[128:]])
    Wlss = jnp.stack([W_ls[:128], W_ls[128:]])
    mu, logstd = _tc3_call(agg3, Wmus, b_mu, Wlss, b_ls)
    return (mu, logstd)


# speedup vs baseline: 6.6275x; 1.0020x over previous
import functools

import jax
import jax.numpy as jnp
from jax import lax
from jax.experimental import pallas as pl
from jax.experimental.pallas import tpu as pltpu
from jax.experimental.pallas import tpu_sc as plsc

N = 10000
E = 320000
NP = 10240
NPH = NP // 2
C = 128
NCH = 2560
E2 = NCH * C
L = 16
NT = 16
ROWS_T = NCH // NT
ROWS_W = NCH // 32
NODES_T = NPH // NT
COLS_T = NP // NT

_MESH = plsc.VectorSubcoreMesh(core_axis_name="c", subcore_axis_name="s")
_f32 = jnp.float32

_SC_CP = pltpu.CompilerParams()
if "needs_layout_passes" in pltpu.CompilerParams.__dataclass_fields__:
    import dataclasses as _dc
    _SC_CP = _dc.replace(_SC_CP, needs_layout_passes=False)


_HR = NP // C


def _deg_body(dst_hbm, ew_hbm, degw_hbm, deg1_hbm,
              dst_v, wv, hw, h1, idx_v, zrow, shw, sh1):
    c = lax.axis_index("c")
    s = lax.axis_index("s")
    wid = c * NT + s
    r0 = wid * ROWS_W
    pltpu.sync_copy(dst_hbm.at[pl.ds(r0, ROWS_W)], dst_v)
    pltpu.sync_copy(ew_hbm.at[pl.ds(r0, ROWS_W)], wv)
    z = jnp.zeros((L,), _f32)
    ones = jnp.ones((L,), _f32)

    @pl.loop(0, _HR)
    def _(j):
        for k in range(C // L):
            sl = pl.ds(k * L, L)
            hw[j, sl] = z
            h1[j, sl] = z

    @pl.loop(0, ROWS_W)
    def _(j):
        for k in range(C // L):
            sl = pl.ds(k * L, L)
            idx = dst_v[j, sl]
            hi = lax.shift_right_logical(idx, 7)
            lo = lax.bitwise_and(idx, 127)
            plsc.addupdate_scatter(hw, [hi, lo], wv[j, sl])
            eid = (r0 + j) * C + k * L
            valid = (lax.iota(jnp.int32, L) + eid) < E
            plsc.addupdate_scatter(h1, [hi, lo], ones, mask=valid)

    @pl.loop(0, 5)
    def _(j):
        for k in range(C // L):
            zrow[j, pl.ds(k * L, L)] = z

    @pl.loop(0, _HR, step=L)
    def _(i):
        idx_v[pl.ds(i, L)] = lax.iota(jnp.int32, L) + i

    a0 = s * 5
    pltpu.sync_copy(zrow, shw.at[pl.ds(a0, 5)])
    pltpu.sync_copy(zrow, sh1.at[pl.ds(a0, 5)])
    plsc.subcore_barrier()
    pltpu.sync_copy(hw, shw.at[idx_v], add=True)
    pltpu.sync_copy(h1, sh1.at[idx_v], add=True)
    plsc.subcore_barrier()

    @pl.when(s < 10)
    def _():
        b0 = s * 8
        pltpu.sync_copy(shw.at[pl.ds(b0, 8)], degw_hbm.at[c, pl.ds(b0, 8)])
        pltpu.sync_copy(sh1.at[pl.ds(b0, 8)], deg1_hbm.at[c, pl.ds(b0, 8)])


@jax.jit
def _deg_call(dst2d, ew2d):
    f = pl.kernel(
        _deg_body,
        out_type=[jax.ShapeDtypeStruct((2, _HR, C), _f32),
                  jax.ShapeDtypeStruct((2, _HR, C), _f32)],
        mesh=_MESH,
        compiler_params=_SC_CP,
        scratch_types=[
            pltpu.VMEM((ROWS_W, C), jnp.int32),
            pltpu.VMEM((ROWS_W, C), _f32),
            pltpu.VMEM((_HR, C), _f32),
            pltpu.VMEM((_HR, C), _f32),
            pltpu.VMEM((_HR,), jnp.int32),
            pltpu.VMEM((5, C), _f32),
            pltpu.VMEM_SHARED((_HR, C), _f32),
            pltpu.VMEM_SHARED((_HR, C), _f32),
        ],
    )
    return f(dst2d, ew2d)


def _coef_body(src_hbm, dst_hbm, ew_hbm, dw_hbm, d1_hbm, cw_hbm, c1_hbm,
               src_v, dst_v, ew_v, tw, t1, ow, o1):
    c = lax.axis_index("c")
    s = lax.axis_index("s")
    wid = c * NT + s
    r0 = wid * ROWS_W
    pltpu.sync_copy(src_hbm.at[pl.ds(r0, ROWS_W)], src_v)
    pltpu.sync_copy(dst_hbm.at[pl.ds(r0, ROWS_W)], dst_v)
    pltpu.sync_copy(ew_hbm.at[pl.ds(r0, ROWS_W)], ew_v)
    pltpu.sync_copy(dw_hbm, tw)
    pltpu.sync_copy(d1_hbm, t1)

    @pl.loop(0, ROWS_W)
    def _(j):
        for k in range(C // L):
            sl = pl.ds(k * L, L)
            si = src_v[j, sl]
            di = dst_v[j, sl]
            ow[j, sl] = (plsc.load_gather(tw, [si]) *
                         plsc.load_gather(tw, [di]) * ew_v[j, sl])
            eid = (r0 + j) * C + k * L
            valid = (lax.iota(jnp.int32, L) + eid) < E
            c1 = plsc.load_gather(t1, [si]) * plsc.load_gather(t1, [di])
            o1[j, sl] = jnp.where(valid, c1, 0.0)

    pltpu.sync_copy(ow, cw_hbm.at[pl.ds(r0, ROWS_W)])
    pltpu.sync_copy(o1, c1_hbm.at[pl.ds(r0, ROWS_W)])


@jax.jit
def _coef_call(src2d, dst2d, ew2d, dinv_w, dinv_1):
    f = pl.kernel(
        _coef_body,
        out_type=[jax.ShapeDtypeStruct((NCH, C), _f32),
                  jax.ShapeDtypeStruct((NCH, C), _f32)],
        mesh=_MESH,
        compiler_params=_SC_CP,
        scratch_types=[
            pltpu.VMEM((ROWS_W, C), jnp.int32),
            pltpu.VMEM((ROWS_W, C), jnp.int32),
            pltpu.VMEM((ROWS_W, C), _f32),
            pltpu.VMEM((NP,), _f32),
            pltpu.VMEM((NP,), _f32),
            pltpu.VMEM((ROWS_W, C), _f32),
            pltpu.VMEM((ROWS_W, C), _f32),
        ],
    )
    return f(src2d, dst2d, ew2d, dinv_w, dinv_1)


ET = ROWS_T * C


def _agg_body(bc, bp, npass, fa, fb, sla, slb, src_hbm, dst_hbm, cf_hbm,
              out_hbm, src1d, dst1d, cf1d, dm, rows, acc):
    c = lax.axis_index("c")
    s = lax.axis_index("s")
    e0 = s * ET
    n0 = s * NODES_T

    def process(feat_hbm, base):
        pltpu.sync_copy(src_hbm.at[pl.ds(e0, ET)], src1d.at[pl.ds(0, ET)])
        pltpu.sync_copy(dst_hbm.at[pl.ds(e0, ET)], dst1d.at[pl.ds(0, ET)])
        pltpu.sync_copy(cf_hbm.at[pl.ds(e0, ET)], cf1d.at[pl.ds(0, ET)])

        def cbody(i, off):
            sl = pl.ds(i * L, L)
            sv = src1d[sl]
            dv = dst1d[sl] - base
            cv = cf1d[sl]
            m = (dv >= 0) & (dv < NPH)
            plsc.store_compressed(src1d.at[pl.ds(off, L)], sv, mask=m)
            plsc.store_compressed(dst1d.at[pl.ds(off, L)], dv, mask=m)
            plsc.store_compressed(cf1d.at[pl.ds(off, L)], cv, mask=m)
            return off + plsc.all_reduce_population_count(m)[0]

        off = lax.fori_loop(0, ET // L, cbody, jnp.int32(0), unroll=4)

        bt = (off // L) * L
        lm = lax.iota(jnp.int32, L) >= (off - bt)
        for ref in (src1d, dst1d):
            v = ref[pl.ds(bt, L)]
            ref[pl.ds(bt, L)] = jnp.where(lm, 0, v)
        vf = cf1d[pl.ds(bt, L)]
        cf1d[pl.ds(bt, L)] = jnp.where(lm, 0.0, vf)
        for k in range(1, 10):
            src1d[pl.ds(bt + k * L, L)] = jnp.zeros((L,), jnp.int32)
            dst1d[pl.ds(bt + k * L, L)] = jnp.zeros((L,), jnp.int32)
            cf1d[pl.ds(bt + k * L, L)] = jnp.zeros((L,), _f32)

        ncht = (off + C - 1) // C

        @pl.loop(0, ROWS_T)
        def _(qj):
            @pl.when(qj < ncht)
            def _():
                for k in range(C // L):
                    dm[0, pl.ds(k * L, L)] = dst1d[pl.ds(qj * C + k * L, L)]
                pltpu.sync_copy(feat_hbm.at[src1d.at[pl.ds(qj * C, C)]], rows)

                @pl.loop(0, C // L)
                def _(g):
                    cvec = cf1d[pl.ds(qj * C + g * L, L)]
                    for l in range(L):
                        sc = cvec[l]
                        i = g * L + l
                        for k in range(128 // L):
                            ksl = pl.ds(k * L, L)
                            rows[i, ksl] = rows[i, ksl] * sc

                pltpu.sync_copy(rows, acc.at[dm.at[0]], add=True)

    for p in range(npass):
        base = (bc * c + bp * p) * NPH

        @pl.when(c == 0)
        def _():
            pltpu.sync_copy(sla.at[pl.ds(base + n0, NODES_T)],
                            acc.at[pl.ds(n0, NODES_T)])

        @pl.when(c == 1)
        def _():
            pltpu.sync_copy(slb.at[pl.ds(base + n0, NODES_T)],
                            acc.at[pl.ds(n0, NODES_T)])

        plsc.subcore_barrier()

        @pl.when(c == 0)
        def _():
            process(fa, base)

        @pl.when(c == 1)
        def _():
            process(fb, base)

        plsc.subcore_barrier()
        pltpu.sync_copy(acc.at[pl.ds(n0, NODES_T)],
                        out_hbm.at[c, pl.ds(base + n0, NODES_T)])


@functools.partial(jax.jit, static_argnums=(0, 1, 2))
def _agg_call(bc, bp, npass, fa, fb, sla, slb, src1, dst1, cf1):
    f = pl.kernel(
        functools.partial(_agg_body, bc, bp, npass),
        out_type=jax.ShapeDtypeStruct((2, NP, 128), _f32),
        mesh=_MESH,
        compiler_params=_SC_CP,
        scratch_types=[
            pltpu.VMEM((ET + 2 * C,), jnp.int32),
            pltpu.VMEM((ET + 2 * C,), jnp.int32),
            pltpu.VMEM((ET + 2 * C,), _f32),
            pltpu.VMEM((1, C), jnp.int32),
            pltpu.VMEM((C, 128), _f32),
            pltpu.VMEM_SHARED((NPH, 128), _f32),
        ],
    )
    return f(fa, fb, sla, slb, src1, dst1, cf1)


_BLK = 1000


def _tc1_body(agg_ref, w_ref, b_ref, dw2_ref, h_ref, sl_ref):
    u = agg_ref[...]
    h = jnp.maximum(jnp.dot(u, w_ref[...],
                            preferred_element_type=_f32) + b_ref[0][None, :], 0.0)
    w2 = dw2_ref[...]
    h_ref[0] = h[:, :128]
    h_ref[1] = h[:, 128:]
    sl_ref[0] = h[:, :128] * w2
    sl_ref[1] = h[:, 128:] * w2


@jax.jit
def _tc1_call(agg1, W1, b1, dw2):
    b1 = b1[None, :]
    return pl.pallas_call(
        _tc1_body,
        out_shape=[jax.ShapeDtypeStruct((2, N, 128), _f32),
                   jax.ShapeDtypeStruct((2, N, 128), _f32)],
        grid=(N // _BLK,),
        in_specs=[
            pl.BlockSpec((_BLK, 128), lambda i: (i, 0)),
            pl.BlockSpec((128, 256), lambda i: (0, 0)),
            pl.BlockSpec((1, 256), lambda i: (0, 0)),
            pl.BlockSpec((_BLK, 1), lambda i: (i, 0)),
        ],
        out_specs=[pl.BlockSpec((2, _BLK, 128), lambda i: (0, i, 0)),
                   pl.BlockSpec((2, _BLK, 128), lambda i: (0, i, 0))],
    )(agg1, W1, b1, dw2)


def _tc2_body(agg_ref, w_ref, b_ref, d12_ref, h_ref, sl_ref):
    h = (jnp.dot(agg_ref[0], w_ref[0], preferred_element_type=_f32) +
         jnp.dot(agg_ref[1], w_ref[1], preferred_element_type=_f32) +
         b_ref[0][None, :])
    w2 = d12_ref[...]
    h_ref[0] = h[:, :128]
    h_ref[1] = h[:, 128:]
    sl_ref[0] = h[:, :128] * w2
    sl_ref[1] = h[:, 128:] * w2


@jax.jit
def _tc2_call(agg2, W2s, b2, d12):
    b2 = b2[None, :]
    return pl.pallas_call(
        _tc2_body,
        out_shape=[jax.ShapeDtypeStruct((2, N, 128), _f32),
                   jax.ShapeDtypeStruct((2, N, 128), _f32)],
        grid=(N // _BLK,),
        in_specs=[
            pl.BlockSpec((2, _BLK, 128), lambda i: (0, i, 0)),
            pl.BlockSpec((2, 128, 256), lambda i: (0, 0, 0)),
            pl.BlockSpec((1, 256), lambda i: (0, 0)),
            pl.BlockSpec((_BLK, 1), lambda i: (i, 0)),
        ],
        out_specs=[pl.BlockSpec((2, _BLK, 128), lambda i: (0, i, 0)),
                   pl.BlockSpec((2, _BLK, 128), lambda i: (0, i, 0))],
    )(agg2, W2s, b2, d12)


def _tc3_body(agg_ref, wmu_ref, bmu_ref, wls_ref, bls_ref, mu_ref, ls_ref):
    a0 = agg_ref[0]
    a1 = agg_ref[1]
    mu_ref[...] = (jnp.dot(a0, wmu_ref[0], preferred_element_type=_f32) +
                   jnp.dot(a1, wmu_ref[1], preferred_element_type=_f32) +
                   bmu_ref[0][None, :])
    ls_ref[...] = (jnp.dot(a0, wls_ref[0], preferred_element_type=_f32) +
                   jnp.dot(a1, wls_ref[1], preferred_element_type=_f32) +
                   bls_ref[0][None, :])


@jax.jit
def _tc3_call(agg3, Wmus, bmu, Wlss, bls):
    bmu = bmu[None, :]
    bls = bls[None, :]
    return pl.pallas_call(
        _tc3_body,
        out_shape=[jax.ShapeDtypeStruct((N, 128), _f32),
                   jax.ShapeDtypeStruct((N, 128), _f32)],
        grid=(N // _BLK,),
        in_specs=[
            pl.BlockSpec((2, _BLK, 128), lambda i: (0, i, 0)),
            pl.BlockSpec((2, 128, 128), lambda i: (0, 0, 0)),
            pl.BlockSpec((1, 128), lambda i: (0, 0)),
            pl.BlockSpec((2, 128, 128), lambda i: (0, 0, 0)),
            pl.BlockSpec((1, 128), lambda i: (0, 0)),
        ],
        out_specs=[pl.BlockSpec((_BLK, 128), lambda i: (i, 0)),
                   pl.BlockSpec((_BLK, 128), lambda i: (i, 0))],
    )(agg3, Wmus, bmu, Wlss, bls)


def kernel(x, edge_index, edge_weight, W1, b1, W2, b2, W_mu, b_mu, W_ls, b_ls):
    src = edge_index[0]
    dst = edge_index[1]
    padn = E2 - E
    zi = jnp.zeros((padn,), jnp.int32)
    src2d = jnp.concatenate([src, zi]).reshape(NCH, C)
    dst2d = jnp.concatenate([dst, zi]).reshape(NCH, C)
    ew2d = jnp.concatenate([edge_weight, jnp.zeros((padn,), _f32)]).reshape(NCH, C)

    degw_p, deg1_p = _deg_call(dst2d, ew2d)
    degw_p = degw_p.reshape(2, NP)
    deg1_p = deg1_p.reshape(2, NP)
    deg_w = degw_p[0] + degw_p[1] + 1.0
    deg_1 = deg1_p[0] + deg1_p[1] + 1.0
    dinv_w = lax.rsqrt(deg_w)
    dinv_1 = lax.rsqrt(deg_1)

    cw2d, c12d = _coef_call(src2d, dst2d, ew2d, dinv_w, dinv_1)

    dw2 = (dinv_w[:N] ** 2)[:, None]
    d12 = (dinv_1[:N] ** 2)[:, None]

    def padrows(a):
        return jnp.concatenate([a, jnp.zeros((NP - N, a.shape[1]), a.dtype)])

    src1 = src2d.reshape(E2)
    dst1 = dst2d.reshape(E2)
    cw1 = cw2d.reshape(E2)
    c11 = c12d.reshape(E2)

    sl1 = padrows(dw2 * x)
    o1 = _agg_call(1, 0, 1, x, x, sl1, sl1, src1, dst1, cw1)
    agg1 = jnp.concatenate([o1[0, :NPH], o1[1, NPH:]])[:N]
    h1, sl2 = _tc1_call(agg1, W1, b1, dw2)

    def agg_256(fh, slh, cf1):
        o = _agg_call(0, 1, 2, fh[0], fh[1], padrows(slh[0]), padrows(slh[1]),
                      src1, dst1, cf1)
        return o[:, :N]

    agg2 = agg_256(h1, sl2, cw1)
    W2s = jnp.stack([W2[:128], W2[128:]])
    h2, sl3 = _tc2_call(agg2, W2s, b2, d12)
    agg3 = agg_256(h2, sl3, c11)
    Wmus = jnp.stack([W_mu[:128], W_mu[128:]])
    Wlss = jnp.stack([W_ls[:128], W_ls[128:]])
    mu, logstd = _tc3_call(agg3, Wmus, b_mu, Wlss, b_ls)
    return (mu, logstd)
